# baseline (speedup 1.0000x reference)
import functools

import jax
import jax.numpy as jnp
from jax import lax
from jax.experimental import pallas as pl
from jax.experimental.pallas import tpu as pltpu
from jax.experimental.pallas import tpu_sc as plsc

N = 10000
E = 320000
D = 128
DW = 144
NC = 2
NS = 16
L = 16
NW = NC * NS
B = 80
CH = 21
NBUF = 3
NB0 = 105
NB1 = 147
TOTB = NS * (NB0 + NB1)
NPAD = 10016
RPT = NPAD // NS
BN = 1000


def _proj_body(x_ref, w_ref, al_ref, ar_ref, fx_ref, er_ref):
    feat = jnp.dot(x_ref[...], w_ref[...], preferred_element_type=jnp.float32)
    el = jnp.dot(feat, al_ref[...], preferred_element_type=jnp.float32)
    er = jnp.dot(feat, ar_ref[...], preferred_element_type=jnp.float32)
    col = lax.broadcasted_iota(jnp.int32, (BN, DW - D), 1)
    tail = jnp.where(col == 0, 1.0, jnp.where(col == 1, el, 0.0))
    fx_ref[...] = jnp.concatenate([feat, tail], axis=1)
    er_ref[...] = er


def _edge_body(featx_hbm, er_hbm, srcb_hbm, dstb_hbm, zer_hbm, acc_hbm,
               src_v, dst_v, ex_v, erb_v, msg_v, acc_sh,
               rsem0, rsem1, rsem2, esem0, esem1, esem2,
               ssem0, ssem1, ssem2):
    rsems = (rsem0, rsem1, rsem2)
    esems = (esem0, esem1, esem2)
    ssems = (ssem0, ssem1, ssem2)
    c = lax.axis_index("c")
    s = lax.axis_index("s")
    base = jnp.where(c == 0, s * NB0, NS * NB0 + s * NB1)
    nstage = jnp.where(c == 0, NB0 // CH, NB1 // CH)

    pltpu.sync_copy(zer_hbm, acc_sh.at[pl.ds(s * RPT, RPT)])
    plsc.subcore_barrier()

    def start_batch(i, b):
        pltpu.async_copy(featx_hbm.at[src_v.at[i]], msg_v.at[b], rsems[b])
        pltpu.async_copy(er_hbm.at[dst_v.at[i]], erb_v.at[b], esems[b])

    def wait_scatter(b):
        pltpu.make_async_copy(
            msg_v.at[b], acc_sh.at[dst_v.at[0]], ssems[b]).wait()

    def do_batch(i, b):
        pltpu.make_async_copy(
            featx_hbm.at[src_v.at[i]], msg_v.at[b], rsems[b]).wait()
        pltpu.make_async_copy(
            er_hbm.at[dst_v.at[i]], erb_v.at[b], esems[b]).wait()

        def group(g, carry):
            rows = lax.iota(jnp.int32, L) + g * L
            elv = plsc.load_gather(
                msg_v, [jnp.full((L,), b, jnp.int32), rows,
                        jnp.full((L,), D + 1, jnp.int32)])
            erv = erb_v[b, pl.ds(g * L, L)]
            sv = elv + erv
            lk = jnp.where(sv >= 0, sv, 0.2 * sv)
            ex_v[pl.ds(g * L, L)] = jnp.exp(lk)
            for k in range(L):
                r = g * L + k
                exb = plsc.load_gather(ex_v, [jnp.zeros((L,), jnp.int32) + r])
                for j in range(DW // L):
                    msg_v[b, r, pl.ds(j * L, L)] = (
                        msg_v[b, r, pl.ds(j * L, L)] * exb)
            return carry

        lax.fori_loop(0, B // L, group, 0)
        pltpu.async_copy(msg_v.at[b], acc_sh.at[dst_v.at[i]], ssems[b],
                         add=True)

        @pl.when(i + NBUF - 1 < CH)
        def _():
            nb = (b + NBUF - 1) % NBUF

            @pl.when(i >= 1)
            def _():
                wait_scatter(nb)

            start_batch(i + NBUF - 1, nb)

    def stage(h, carry):
        @pl.when(h > 0)
        def _():
            for b in range(NBUF):
                wait_scatter(b)

        pltpu.sync_copy(srcb_hbm.at[pl.ds(base + h * CH, CH)], src_v)
        pltpu.sync_copy(dstb_hbm.at[pl.ds(base + h * CH, CH)], dst_v)
        start_batch(0, 0)
        start_batch(1, 1)

        def triple(t, carry2):
            do_batch(NBUF * t, 0)
            do_batch(NBUF * t + 1, 1)
            do_batch(NBUF * t + 2, 2)
            return carry2

        lax.fori_loop(0, CH // NBUF, triple, 0)
        return carry

    lax.fori_loop(0, nstage, stage, 0)
    for b in range(NBUF):
        wait_scatter(b)

    plsc.subcore_barrier()
    pltpu.sync_copy(acc_sh.at[pl.ds(s * RPT, RPT)],
                    acc_hbm.at[c, pl.ds(s * RPT, RPT)])


def _make_edge_kernel():
    return functools.partial(
        pl.kernel,
        out_type=jax.ShapeDtypeStruct((NC, NPAD, DW), jnp.float32),
        mesh=plsc.VectorSubcoreMesh(core_axis_name="c", subcore_axis_name="s",
                                    num_cores=NC, num_subcores=NS),
        scratch_types=[
            pltpu.VMEM((CH, B), jnp.int32),
            pltpu.VMEM((CH, B), jnp.int32),
            pltpu.VMEM((B,), jnp.float32),
            pltpu.VMEM((NBUF, B), jnp.float32),
            pltpu.VMEM((NBUF, B, DW), jnp.float32),
            pltpu.VMEM_SHARED((NPAD, DW), jnp.float32),
        ] + [pltpu.SemaphoreType.DMA] * (3 * NBUF),
        compiler_params=pltpu.CompilerParams(
            needs_layout_passes=False, use_tc_tiling_on_sc=False),
    )(_edge_body)


def _final_body(acc_ref, bias_ref, out_ref):
    a = acc_ref[0] + acc_ref[1]
    m = a[:, :D]
    dn = a[:, D:D + 1]
    dn = jnp.where(dn > 0, dn, 1.0)
    out_ref[...] = jnp.tanh(m / dn + bias_ref[...])


def kernel(x, edge_index, W, attn_l, attn_r, bias):
    featx, er = pl.pallas_call(
        _proj_body,
        grid=(N // BN,),
        in_specs=[
            pl.BlockSpec((BN, D), lambda i: (i, 0)),
            pl.BlockSpec((D, D), lambda i: (0, 0)),
            pl.BlockSpec((D, 1), lambda i: (0, 0)),
            pl.BlockSpec((D, 1), lambda i: (0, 0)),
        ],
        out_specs=[
            pl.BlockSpec((BN, DW), lambda i: (i, 0)),
            pl.BlockSpec((BN, 1), lambda i: (i, 0)),
        ],
        out_shape=[
            jax.ShapeDtypeStruct((N, DW), jnp.float32),
            jax.ShapeDtypeStruct((N, 1), jnp.float32),
        ],
    )(x, W, attn_l.reshape(D, 1), attn_r.reshape(D, 1))

    er_pad = jnp.concatenate(
        [er.reshape(N), jnp.zeros((NPAD - N,), jnp.float32)])
    pad = TOTB * B - E
    srcb = jnp.concatenate(
        [edge_index[0], jnp.zeros((pad,), jnp.int32)]).reshape(TOTB, B)
    dstb = jnp.concatenate(
        [edge_index[1], jnp.full((pad,), N, jnp.int32)]).reshape(TOTB, B)
    zer = jnp.zeros((RPT, DW), jnp.float32)

    acc = _make_edge_kernel()(featx, er_pad, srcb, dstb, zer)

    out = pl.pallas_call(
        _final_body,
        grid=(N // BN,),
        in_specs=[
            pl.BlockSpec((NC, BN, DW), lambda i: (0, i, 0)),
            pl.BlockSpec((1, D), lambda i: (0, 0)),
        ],
        out_specs=pl.BlockSpec((BN, D), lambda i: (i, 0)),
        out_shape=jax.ShapeDtypeStruct((N, D), jnp.float32),
    )(acc, bias.reshape(1, D))
    return out


# speedup vs baseline: 1.1028x; 1.1028x over previous
import functools

import jax
import jax.numpy as jnp
from jax import lax
from jax.experimental import pallas as pl
from jax.experimental.pallas import tpu as pltpu
from jax.experimental.pallas import tpu_sc as plsc

N = 10000
E = 320000
D = 128
DW = 144
NC = 2
NS = 16
L = 16
NW = NC * NS
B = 80
CH = 21
NBUF = 3
NB0 = 147
NB1 = 105
TOTB = NS * (NB0 + NB1)
NPAD = 10016
RPT = NPAD // NS
BN = 1000


def _proj_body(x_ref, w_ref, al_ref, ar_ref, fx_ref, er_ref):
    feat = jnp.dot(x_ref[...], w_ref[...], preferred_element_type=jnp.float32)
    el = jnp.dot(feat, al_ref[...], preferred_element_type=jnp.float32)
    er = jnp.dot(feat, ar_ref[...], preferred_element_type=jnp.float32)
    col = lax.broadcasted_iota(jnp.int32, (BN, DW - D), 1)
    tail = jnp.where(col == 0, 1.0, jnp.where(col == 1, el, 0.0))
    fx_ref[...] = jnp.concatenate([feat, tail], axis=1)
    er_ref[...] = er


def _edge_body(featx_hbm, er_hbm, srcb_hbm, dstb_hbm, zer_hbm, acc_hbm,
               src_v, dst_v, ex_v, erb_v, msg_v, acc_sh,
               rsem0, rsem1, rsem2, esem0, esem1, esem2,
               ssem0, ssem1, ssem2):
    rsems = (rsem0, rsem1, rsem2)
    esems = (esem0, esem1, esem2)
    ssems = (ssem0, ssem1, ssem2)
    c = lax.axis_index("c")
    s = lax.axis_index("s")
    base = jnp.where(c == 0, s * NB0, NS * NB0 + s * NB1)
    nstage = jnp.where(c == 0, NB0 // CH, NB1 // CH)

    pltpu.sync_copy(zer_hbm, acc_sh.at[pl.ds(s * RPT, RPT)])
    plsc.subcore_barrier()

    def start_batch(i, b):
        pltpu.async_copy(featx_hbm.at[src_v.at[i]], msg_v.at[b], rsems[b])
        pltpu.async_copy(er_hbm.at[dst_v.at[i]], erb_v.at[b], esems[b])

    def wait_scatter(b):
        pltpu.make_async_copy(
            msg_v.at[b], acc_sh.at[dst_v.at[0]], ssems[b]).wait()

    def do_batch(i, b):
        pltpu.make_async_copy(
            featx_hbm.at[src_v.at[i]], msg_v.at[b], rsems[b]).wait()
        pltpu.make_async_copy(
            er_hbm.at[dst_v.at[i]], erb_v.at[b], esems[b]).wait()

        def group(g, carry):
            rows = lax.iota(jnp.int32, L) + g * L
            elv = plsc.load_gather(
                msg_v, [jnp.full((L,), b, jnp.int32), rows,
                        jnp.full((L,), D + 1, jnp.int32)])
            erv = erb_v[b, pl.ds(g * L, L)]
            sv = elv + erv
            lk = jnp.where(sv >= 0, sv, 0.2 * sv)
            ex_v[pl.ds(g * L, L)] = jnp.exp(lk)
            for k in range(L):
                r = g * L + k
                exb = plsc.load_gather(ex_v, [jnp.zeros((L,), jnp.int32) + r])
                for j in range(DW // L):
                    msg_v[b, r, pl.ds(j * L, L)] = (
                        msg_v[b, r, pl.ds(j * L, L)] * exb)
            return carry

        lax.fori_loop(0, B // L, group, 0)
        pltpu.async_copy(msg_v.at[b], acc_sh.at[dst_v.at[i]], ssems[b],
                         add=True)

        @pl.when(i + NBUF - 1 < CH)
        def _():
            nb = (b + NBUF - 1) % NBUF

            @pl.when(i >= 1)
            def _():
                wait_scatter(nb)

            start_batch(i + NBUF - 1, nb)

    def stage(h, carry):
        @pl.when(h > 0)
        def _():
            for b in range(NBUF):
                wait_scatter(b)

        pltpu.sync_copy(srcb_hbm.at[pl.ds(base + h * CH, CH)], src_v)
        pltpu.sync_copy(dstb_hbm.at[pl.ds(base + h * CH, CH)], dst_v)
        start_batch(0, 0)
        start_batch(1, 1)

        def triple(t, carry2):
            do_batch(NBUF * t, 0)
            do_batch(NBUF * t + 1, 1)
            do_batch(NBUF * t + 2, 2)
            return carry2

        lax.fori_loop(0, CH // NBUF, triple, 0)
        return carry

    lax.fori_loop(0, nstage, stage, 0)
    for b in range(NBUF):
        wait_scatter(b)

    plsc.subcore_barrier()
    pltpu.sync_copy(acc_sh.at[pl.ds(s * RPT, RPT)],
                    acc_hbm.at[c, pl.ds(s * RPT, RPT)])


def _make_edge_kernel():
    return functools.partial(
        pl.kernel,
        out_type=jax.ShapeDtypeStruct((NC, NPAD, DW), jnp.float32),
        mesh=plsc.VectorSubcoreMesh(core_axis_name="c", subcore_axis_name="s",
                                    num_cores=NC, num_subcores=NS),
        scratch_types=[
            pltpu.VMEM((CH, B), jnp.int32),
            pltpu.VMEM((CH, B), jnp.int32),
            pltpu.VMEM((B,), jnp.float32),
            pltpu.VMEM((NBUF, B), jnp.float32),
            pltpu.VMEM((NBUF, B, DW), jnp.float32),
            pltpu.VMEM_SHARED((NPAD, DW), jnp.float32),
        ] + [pltpu.SemaphoreType.DMA] * (3 * NBUF),
        compiler_params=pltpu.CompilerParams(
            needs_layout_passes=False, use_tc_tiling_on_sc=False),
    )(_edge_body)


def _final_body(acc_ref, bias_ref, out_ref):
    a = acc_ref[0] + acc_ref[1]
    m = a[:, :D]
    dn = a[:, D:D + 1]
    dn = jnp.where(dn > 0, dn, 1.0)
    out_ref[...] = jnp.tanh(m / dn + bias_ref[...])


def kernel(x, edge_index, W, attn_l, attn_r, bias):
    featx, er = pl.pallas_call(
        _proj_body,
        grid=(N // BN,),
        in_specs=[
            pl.BlockSpec((BN, D), lambda i: (i, 0)),
            pl.BlockSpec((D, D), lambda i: (0, 0)),
            pl.BlockSpec((D, 1), lambda i: (0, 0)),
            pl.BlockSpec((D, 1), lambda i: (0, 0)),
        ],
        out_specs=[
            pl.BlockSpec((BN, DW), lambda i: (i, 0)),
            pl.BlockSpec((BN, 1), lambda i: (i, 0)),
        ],
        out_shape=[
            jax.ShapeDtypeStruct((N, DW), jnp.float32),
            jax.ShapeDtypeStruct((N, 1), jnp.float32),
        ],
    )(x, W, attn_l.reshape(D, 1), attn_r.reshape(D, 1))

    er_pad = jnp.concatenate(
        [er.reshape(N), jnp.zeros((NPAD - N,), jnp.float32)])
    pad = TOTB * B - E
    srcb = jnp.concatenate(
        [edge_index[0], jnp.zeros((pad,), jnp.int32)]).reshape(TOTB, B)
    dstb = jnp.concatenate(
        [edge_index[1], jnp.full((pad,), N, jnp.int32)]).reshape(TOTB, B)
    zer = jnp.zeros((RPT, DW), jnp.float32)

    acc = _make_edge_kernel()(featx, er_pad, srcb, dstb, zer)

    out = pl.pallas_call(
        _final_body,
        grid=(N // BN,),
        in_specs=[
            pl.BlockSpec((NC, BN, DW), lambda i: (0, i, 0)),
            pl.BlockSpec((1, D), lambda i: (0, 0)),
        ],
        out_specs=pl.BlockSpec((BN, D), lambda i: (i, 0)),
        out_shape=jax.ShapeDtypeStruct((N, D), jnp.float32),
    )(acc, bias.reshape(1, D))
    return out
